# Initial kernel scaffold; baseline (speedup 1.0000x reference)
import jax
import jax.numpy as jnp
from jax.experimental import pallas as pl


def kernel(cosine, labels):
    raise NotImplementedError("write your pallas kernel here")


# speedup vs baseline: 2.2738x; 2.2738x over previous
import functools
import math

import jax
import jax.numpy as jnp
from jax.experimental import pallas as pl
from jax.experimental.pallas import tpu as pltpu

_S = 64.0
_M = 0.5
_COS_M = math.cos(_M)
_SIN_M = math.sin(_M)


def _row_kernel(labels_ref, cos_ref, out_ref, acc_sum, acc_picked, *, bc, n):
    c = pl.program_id(1)
    nc = pl.num_programs(1)

    @pl.when(c == 0)
    def _():
        acc_sum[...] = jnp.zeros_like(acc_sum)
        acc_picked[...] = jnp.zeros_like(acc_picked)

    x = cos_ref[...]
    cols = c * bc + jax.lax.broadcasted_iota(jnp.int32, x.shape, 1)
    valid = cols < n
    e = jnp.exp(jnp.where(valid, x * _S - _S, -1e30))
    acc_sum[...] += jnp.sum(e, axis=1, keepdims=True)
    lab = labels_ref[...]
    hit = cols == lab
    acc_picked[...] += jnp.sum(jnp.where(hit, x, 0.0), axis=1, keepdims=True)

    @pl.when(c == nc - 1)
    def _():
        p = acc_picked[...]
        m = p * _COS_M - jnp.sqrt(jnp.maximum(1.0 - p * p, 0.0)) * _SIN_M
        s = acc_sum[...] - jnp.exp(p * _S - _S) + jnp.exp(m * _S - _S)
        out_ref[...] = (_S + jnp.log(s)) - m * _S


def _mean_kernel(x_ref, o_ref):
    o_ref[...] = jnp.sum(x_ref[...], axis=(0, 1), keepdims=True) / x_ref.shape[0]


def kernel(cosine, labels):
    if labels.ndim == 2:
        labels = labels.squeeze(1)
    b, n = cosine.shape
    labels2 = labels.astype(jnp.int32).reshape(b, 1)
    br, bc = 256, 2048
    grid = (b // br, pl.cdiv(n, bc))
    row_losses = pl.pallas_call(
        functools.partial(_row_kernel, bc=bc, n=n),
        grid=grid,
        in_specs=[
            pl.BlockSpec((br, 1), lambda r, c: (r, 0)),
            pl.BlockSpec((br, bc), lambda r, c: (r, c)),
        ],
        out_specs=pl.BlockSpec((br, 1), lambda r, c: (r, 0)),
        out_shape=jax.ShapeDtypeStruct((b, 1), jnp.float32),
        scratch_shapes=[
            pltpu.VMEM((br, 1), jnp.float32),
            pltpu.VMEM((br, 1), jnp.float32),
        ],
        compiler_params=pltpu.CompilerParams(
            dimension_semantics=("parallel", "arbitrary")
        ),
    )(labels2, cosine)
    loss = pl.pallas_call(
        _mean_kernel,
        out_shape=jax.ShapeDtypeStruct((1, 1), jnp.float32),
    )(row_losses)
    return loss.reshape(())


# speedup vs baseline: 2.6211x; 1.1528x over previous
import functools
import math

import jax
import jax.numpy as jnp
from jax.experimental import pallas as pl
from jax.experimental.pallas import tpu as pltpu

_S = 64.0
_M = 0.5
_COS_M = math.cos(_M)
_SIN_M = math.sin(_M)
_LOG2E = math.log2(math.e)


def _row_kernel(labels_ref, cos_ref, out_ref, acc_sum, acc_picked, *, bc, n):
    c = pl.program_id(1)
    nc = pl.num_programs(1)

    @pl.when(c == 0)
    def _():
        acc_sum[...] = jnp.zeros_like(acc_sum)
        acc_picked[...] = jnp.zeros_like(acc_picked)

    x = cos_ref[...]
    cols = c * bc + jax.lax.broadcasted_iota(jnp.int32, x.shape, 1)
    lab = labels_ref[...]
    hit = cols == lab
    k = _S * _LOG2E

    @pl.when(c < nc - 1)
    def _():
        e = jnp.exp2(x * k - k)
        acc_sum[...] += jnp.sum(e, axis=1, keepdims=True)

    @pl.when(c == nc - 1)
    def _():
        e = jnp.exp2(jnp.where(cols < n, x * k - k, -1e30))
        acc_sum[...] += jnp.sum(e, axis=1, keepdims=True)

    acc_picked[...] += jnp.sum(jnp.where(hit, x, 0.0), axis=1, keepdims=True)

    @pl.when(c == nc - 1)
    def _():
        p = acc_picked[...]
        m = p * _COS_M - jnp.sqrt(jnp.maximum(1.0 - p * p, 0.0)) * _SIN_M
        s = acc_sum[...] - jnp.exp2(p * k - k) + jnp.exp2(m * k - k)
        out_ref[...] = (_S + jnp.log(s)) - m * _S


def _mean_kernel(x_ref, o_ref):
    o_ref[...] = jnp.sum(x_ref[...], axis=(0, 1), keepdims=True) / x_ref.shape[0]


def kernel(cosine, labels):
    if labels.ndim == 2:
        labels = labels.squeeze(1)
    b, n = cosine.shape
    labels2 = labels.astype(jnp.int32).reshape(b, 1)
    br, bc = 256, 8192
    grid = (b // br, pl.cdiv(n, bc))
    row_losses = pl.pallas_call(
        functools.partial(_row_kernel, bc=bc, n=n),
        grid=grid,
        in_specs=[
            pl.BlockSpec((br, 1), lambda r, c: (r, 0)),
            pl.BlockSpec((br, bc), lambda r, c: (r, c)),
        ],
        out_specs=pl.BlockSpec((br, 1), lambda r, c: (r, 0)),
        out_shape=jax.ShapeDtypeStruct((b, 1), jnp.float32),
        scratch_shapes=[
            pltpu.VMEM((br, 1), jnp.float32),
            pltpu.VMEM((br, 1), jnp.float32),
        ],
        compiler_params=pltpu.CompilerParams(
            dimension_semantics=("parallel", "arbitrary")
        ),
    )(labels2, cosine)
    loss = pl.pallas_call(
        _mean_kernel,
        out_shape=jax.ShapeDtypeStruct((1, 1), jnp.float32),
    )(row_losses)
    return loss.reshape(())


# speedup vs baseline: 7.6047x; 2.9013x over previous
import functools
import math

import jax
import jax.numpy as jnp
from jax.experimental import pallas as pl
from jax.experimental.pallas import tpu as pltpu

_S = 64.0
_M = 0.5
_COS_M = math.cos(_M)
_SIN_M = math.sin(_M)
_LOG2E = math.log2(math.e)


def _arc_kernel(labels_ref, xt_ref, out_ref, acc_sum, acc_picked, *, bn, n):
    c = pl.program_id(0)
    nc = pl.num_programs(0)
    k = _S * _LOG2E

    @pl.when(c == 0)
    def _():
        acc_sum[...] = jnp.zeros_like(acc_sum)
        acc_picked[...] = jnp.zeros_like(acc_picked)

    x = xt_ref[...]
    rows = c * bn + jax.lax.broadcasted_iota(jnp.int32, x.shape, 0)
    lab = labels_ref[...]
    hit = rows == lab

    @pl.when(c < nc - 1)
    def _():
        acc_sum[...] += jnp.sum(jnp.exp2(x * k - k), axis=0, keepdims=True)

    @pl.when(c == nc - 1)
    def _():
        e = jnp.exp2(jnp.where(rows < n, x * k - k, -1e30))
        acc_sum[...] += jnp.sum(e, axis=0, keepdims=True)

    acc_picked[...] += jnp.sum(jnp.where(hit, x, 0.0), axis=0, keepdims=True)

    @pl.when(c == nc - 1)
    def _():
        p = acc_picked[...]
        m = p * _COS_M - jnp.sqrt(jnp.maximum(1.0 - p * p, 0.0)) * _SIN_M
        s = acc_sum[...] - jnp.exp2(p * k - k) + jnp.exp2(m * k - k)
        loss = (_S + jnp.log(s)) - m * _S
        out_ref[...] = jnp.sum(loss, axis=1, keepdims=True) / loss.shape[1]


def kernel(cosine, labels):
    if labels.ndim == 2:
        labels = labels.squeeze(1)
    b, n = cosine.shape
    xt = cosine.T
    labels2 = labels.astype(jnp.int32).reshape(1, b)
    bn = 2048
    grid = (pl.cdiv(n, bn),)
    loss = pl.pallas_call(
        functools.partial(_arc_kernel, bn=bn, n=n),
        grid=grid,
        in_specs=[
            pl.BlockSpec((1, b), lambda c: (0, 0)),
            pl.BlockSpec((bn, b), lambda c: (c, 0)),
        ],
        out_specs=pl.BlockSpec((1, 1), lambda c: (0, 0)),
        out_shape=jax.ShapeDtypeStruct((1, 1), jnp.float32),
        scratch_shapes=[
            pltpu.VMEM((1, b), jnp.float32),
            pltpu.VMEM((1, b), jnp.float32),
        ],
    )(labels2, xt)
    return loss.reshape(())


# speedup vs baseline: 7.6769x; 1.0095x over previous
import functools
import math

import jax
import jax.numpy as jnp
from jax.experimental import pallas as pl
from jax.experimental.pallas import tpu as pltpu

_S = 64.0
_M = 0.5
_COS_M = math.cos(_M)
_SIN_M = math.sin(_M)
_LOG2E = math.log2(math.e)


def _arc_kernel(labels_ref, xt_ref, out_ref, acc_sum, acc_picked, *, bn, n):
    c = pl.program_id(0)
    nc = pl.num_programs(0)
    k = _S * _LOG2E

    @pl.when(c == 0)
    def _():
        acc_sum[...] = jnp.zeros_like(acc_sum)
        acc_picked[...] = jnp.zeros_like(acc_picked)

    x = xt_ref[...]
    rows_local = jax.lax.broadcasted_iota(jnp.int32, x.shape, 0)
    lab_adj = labels_ref[...] - c * bn
    hit = rows_local == lab_adj

    @pl.when(c < nc - 1)
    def _():
        acc_sum[...] += jnp.sum(jnp.exp2(x * k - k), axis=0, keepdims=True)

    @pl.when(c == nc - 1)
    def _():
        e = jnp.exp2(jnp.where(rows_local < n - c * bn, x * k - k, -1e30))
        acc_sum[...] += jnp.sum(e, axis=0, keepdims=True)

    acc_picked[...] += jnp.sum(jnp.where(hit, x, 0.0), axis=0, keepdims=True)

    @pl.when(c == nc - 1)
    def _():
        p = acc_picked[...]
        m = p * _COS_M - jnp.sqrt(jnp.maximum(1.0 - p * p, 0.0)) * _SIN_M
        s = acc_sum[...] - jnp.exp2(p * k - k) + jnp.exp2(m * k - k)
        loss = (_S + jnp.log(s)) - m * _S
        out_ref[...] = jnp.sum(loss, axis=1, keepdims=True) / loss.shape[1]


def kernel(cosine, labels):
    if labels.ndim == 2:
        labels = labels.squeeze(1)
    b, n = cosine.shape
    xt = cosine.T
    labels2 = labels.astype(jnp.int32).reshape(1, b)
    bn = 2048
    grid = (pl.cdiv(n, bn),)
    loss = pl.pallas_call(
        functools.partial(_arc_kernel, bn=bn, n=n),
        grid=grid,
        in_specs=[
            pl.BlockSpec((1, b), lambda c: (0, 0)),
            pl.BlockSpec((bn, b), lambda c: (c, 0)),
        ],
        out_specs=pl.BlockSpec((1, 1), lambda c: (0, 0)),
        out_shape=jax.ShapeDtypeStruct((1, 1), jnp.float32),
        scratch_shapes=[
            pltpu.VMEM((1, b), jnp.float32),
            pltpu.VMEM((1, b), jnp.float32),
        ],
    )(labels2, xt)
    return loss.reshape(())
